# Initial kernel scaffold; baseline (speedup 1.0000x reference)
import functools

import jax
import jax.numpy as jnp
from jax import lax
from jax.experimental import pallas as pl
from jax.experimental.pallas import tpu as pltpu
from jax.experimental.pallas import tpu_sc as plsc

NC = 2
NS = 16
L = 16


def _sc_segment_sum(x, src, dst, w, n_rows):
    e_total = src.shape[0]
    d = x.shape[1]
    nw = NC * NS
    epw = e_total // nw
    chunk = 80
    n_chunks = epw // chunk
    assert epw % chunk == 0 and n_rows % NS == 0
    rows_per_tile = n_rows // NS

    mesh = plsc.VectorSubcoreMesh(core_axis_name="c", subcore_axis_name="s")

    @functools.partial(
        pl.kernel,
        out_type=jax.ShapeDtypeStruct((NC, n_rows, d), jnp.float32),
        mesh=mesh,
        scratch_types=[
            pltpu.VMEM((chunk,), jnp.int32),
            pltpu.VMEM((chunk,), jnp.int32),
            pltpu.VMEM((chunk,), jnp.float32),
            pltpu.VMEM((chunk, 128), jnp.float32),
            pltpu.VMEM_SHARED((10000, 128), jnp.float32),
            pltpu.SemaphoreType.DMA,
        ],
    )
    def sc_kernel(x_hbm, src_hbm, dst_hbm, w_hbm, part_hbm,
                  si_v, di_v, w_v, rows_v, acc_sh, sem):
        cidx = lax.axis_index("c")
        sidx = lax.axis_index("s")
        wid = sidx * NC + cidx
        base = wid * epw

        zero = jnp.zeros((L,), jnp.float32)

        @pl.loop(0, chunk)
        def _(i):
            for j in range(d // L):
                rows_v[i, pl.ds(j * L, L)] = zero

        row0 = sidx * rows_per_tile
        full, rem = divmod(rows_per_tile, chunk)
        for k in range(full):
            pltpu.sync_copy(rows_v, acc_sh.at[pl.ds(row0 + k * chunk, chunk)])
        if rem:
            pltpu.sync_copy(rows_v.at[pl.ds(0, rem)],
                            acc_sh.at[pl.ds(row0 + full * chunk, rem)])
        plsc.subcore_barrier()

        @pl.loop(0, n_chunks)
        def _(k):
            off = base + k * chunk
            pltpu.sync_copy(src_hbm.at[pl.ds(off, chunk)], si_v)
            pltpu.sync_copy(dst_hbm.at[pl.ds(off, chunk)], di_v)
            pltpu.sync_copy(w_hbm.at[pl.ds(off, chunk)], w_v)
            pltpu.async_copy(x_hbm.at[si_v], rows_v, sem).wait()

            @pl.loop(0, chunk)
            def _(e):
                wv = plsc.load_gather(w_v, [jnp.full((L,), e, jnp.int32)])
                for j in range(d // L):
                    sl = pl.ds(j * L, L)
                    rows_v[e, sl] = rows_v[e, sl] * wv

            pltpu.sync_copy(rows_v, acc_sh.at[di_v], add=True)

        plsc.subcore_barrier()
        pltpu.sync_copy(acc_sh.at[pl.ds(row0, rows_per_tile)],
                        part_hbm.at[cidx, pl.ds(row0, rows_per_tile)])

    return sc_kernel(x, src, dst, w)


def _tc_combine_matmul_relu(partials, W):
    n_rows, d_in = partials.shape[1], partials.shape[2]
    d_out = W.shape[1]
    blk = 1000

    def body(p_ref, w_ref, o_ref):
        p = p_ref[0] + p_ref[1]
        o_ref[...] = jnp.maximum(
            jnp.dot(p, w_ref[...], preferred_element_type=jnp.float32), 0.0)

    return pl.pallas_call(
        body,
        grid=(n_rows // blk,),
        in_specs=[
            pl.BlockSpec((NC, blk, d_in), lambda i: (0, i, 0)),
            pl.BlockSpec((d_in, d_out), lambda i: (0, 0)),
        ],
        out_specs=pl.BlockSpec((blk, d_out), lambda i: (i, 0)),
        out_shape=jax.ShapeDtypeStruct((n_rows, d_out), jnp.float32),
    )(partials, W)


def kernel(x, edge_index, edge_weight, W):
    n_rows = x.shape[0]
    dst = edge_index[0]
    src = edge_index[1]
    partials = _sc_segment_sum(x, src, dst, edge_weight, n_rows)
    return _tc_combine_matmul_relu(partials, W)


# speedup vs baseline: 4.0792x; 4.0792x over previous
import dataclasses
import functools

import jax
import jax.numpy as jnp
from jax import lax
from jax.experimental import pallas as pl
from jax.experimental.pallas import tpu as pltpu
from jax.experimental.pallas import tpu_sc as plsc

NC = 2
NS = 16
L = 16


def _sc_segment_sum(x, src, dst, w, n_rows):
    e_total = src.shape[0]
    d = x.shape[1]
    nw = NC * NS
    epw = e_total // nw
    chunk = 80
    n_chunks = epw // chunk
    assert epw % chunk == 0
    tile_step = (n_rows // NS) // 8 * 8
    tile_win = n_rows - tile_step * (NS - 1)
    assert tile_win % chunk == 0 and tile_win >= tile_step

    mesh = plsc.VectorSubcoreMesh(core_axis_name="c", subcore_axis_name="s")
    cp = pltpu.CompilerParams()
    if "needs_layout_passes" in pltpu.CompilerParams.__dataclass_fields__:
        cp = dataclasses.replace(cp, needs_layout_passes=False)

    @functools.partial(
        pl.kernel,
        out_type=jax.ShapeDtypeStruct((NC, n_rows, d), jnp.float32),
        mesh=mesh,
        compiler_params=cp,
        scratch_types=[
            pltpu.VMEM((chunk,), jnp.int32),
            pltpu.VMEM((chunk,), jnp.int32),
            pltpu.VMEM((chunk,), jnp.float32),
            pltpu.VMEM((chunk, d), jnp.float32),
            pltpu.VMEM_SHARED((n_rows, d), jnp.float32),
            pltpu.SemaphoreType.DMA,
        ],
    )
    def sc_kernel(x_hbm, src_hbm, dst_hbm, w_hbm, part_hbm,
                  si_v, di_v, w_v, rows_v, acc_sh, sem):
        cidx = lax.axis_index("c")
        sidx = lax.axis_index("s")
        wid = sidx * NC + cidx
        base = wid * epw

        zero = jnp.zeros((L,), jnp.float32)

        @pl.loop(0, chunk)
        def _(i):
            for j in range(d // L):
                rows_v[i, pl.ds(j * L, L)] = zero

        row0 = sidx * tile_step
        for k in range(tile_win // chunk):
            pltpu.sync_copy(rows_v, acc_sh.at[pl.ds(row0 + k * chunk, chunk)])
        plsc.subcore_barrier()

        @pl.loop(0, n_chunks)
        def _(k):
            off = base + k * chunk
            pltpu.sync_copy(src_hbm.at[pl.ds(off, chunk)], si_v)
            pltpu.sync_copy(dst_hbm.at[pl.ds(off, chunk)], di_v)
            pltpu.sync_copy(w_hbm.at[pl.ds(off, chunk)], w_v)
            pltpu.async_copy(x_hbm.at[si_v], rows_v, sem).wait()

            @pl.loop(0, chunk)
            def _(e):
                wv = plsc.load_gather(w_v, [jnp.full((L,), e, jnp.int32)])
                for j in range(d // L):
                    sl = pl.ds(j * L, L)
                    rows_v[e, sl] = rows_v[e, sl] * wv

            pltpu.sync_copy(rows_v, acc_sh.at[di_v], add=True)

        plsc.subcore_barrier()
        pltpu.sync_copy(acc_sh.at[pl.ds(row0, tile_win)],
                        part_hbm.at[cidx, pl.ds(row0, tile_win)])

    return sc_kernel(x, src, dst, w)


def _tc_combine_matmul_relu(partials, W):
    n_rows, d_in = partials.shape[1], partials.shape[2]
    d_out = W.shape[1]
    blk = 1000

    def body(p_ref, w_ref, o_ref):
        p = p_ref[0] + p_ref[1]
        o_ref[...] = jnp.maximum(
            jnp.dot(p, w_ref[...], preferred_element_type=jnp.float32), 0.0)

    return pl.pallas_call(
        body,
        grid=(n_rows // blk,),
        in_specs=[
            pl.BlockSpec((NC, blk, d_in), lambda i: (0, i, 0)),
            pl.BlockSpec((d_in, d_out), lambda i: (0, 0)),
        ],
        out_specs=pl.BlockSpec((blk, d_out), lambda i: (i, 0)),
        out_shape=jax.ShapeDtypeStruct((n_rows, d_out), jnp.float32),
    )(partials, W)


def kernel(x, edge_index, edge_weight, W):
    n_rows = x.shape[0]
    dst = edge_index[0]
    src = edge_index[1]
    partials = _sc_segment_sum(x, src, dst, edge_weight, n_rows)
    return _tc_combine_matmul_relu(partials, W)


# speedup vs baseline: 9.2497x; 2.2675x over previous
import dataclasses
import functools

import jax
import jax.numpy as jnp
from jax import lax
from jax.experimental import pallas as pl
from jax.experimental.pallas import tpu as pltpu
from jax.experimental.pallas import tpu_sc as plsc

NC = 2
NS = 16
L = 16


def _sc_segment_sum(x, src, dst3, w, n_rows):
    nw = NC * NS
    e_total = src.shape[0]
    d = x.shape[1]
    epw = e_total // nw
    chunk = 80
    n_chunks = epw // chunk
    assert epw % chunk == 0 and n_chunks % 2 == 1 and dst3.shape == (nw, n_chunks, chunk)
    tile_step = (n_rows // NS) // 8 * 8
    tile_win = n_rows - tile_step * (NS - 1)
    assert tile_win % chunk == 0 and tile_win >= tile_step

    mesh = plsc.VectorSubcoreMesh(core_axis_name="c", subcore_axis_name="s")
    cp = pltpu.CompilerParams()
    if "needs_layout_passes" in pltpu.CompilerParams.__dataclass_fields__:
        cp = dataclasses.replace(cp, needs_layout_passes=False)

    @functools.partial(
        pl.kernel,
        out_type=jax.ShapeDtypeStruct((NC, n_rows, d), jnp.float32),
        mesh=mesh,
        compiler_params=cp,
        scratch_types=[
            pltpu.VMEM((n_chunks, chunk), jnp.int32),
            pltpu.VMEM((epw,), jnp.float32),
            pltpu.VMEM((chunk,), jnp.int32),
            pltpu.VMEM((chunk,), jnp.int32),
            pltpu.VMEM((chunk, d), jnp.float32),
            pltpu.VMEM((chunk, d), jnp.float32),
            pltpu.VMEM_SHARED((n_rows, d), jnp.float32),
            pltpu.SemaphoreType.DMA,
            pltpu.SemaphoreType.DMA,
            pltpu.SemaphoreType.DMA,
            pltpu.SemaphoreType.DMA,
            pltpu.SemaphoreType.DMA,
            pltpu.SemaphoreType.DMA,
        ],
    )
    def sc_kernel(x_hbm, src_hbm, dst3_hbm, w_hbm, part_hbm,
                  di_all, w_all, si_c0, si_c1, rows0, rows1, acc_sh,
                  sg0, sg1, ss0, ss1, sl0, sl1):
        cidx = lax.axis_index("c")
        sidx = lax.axis_index("s")
        wid = sidx * NC + cidx
        base = wid * epw

        c2 = pltpu.async_copy(dst3_hbm.at[wid], di_all, sg1)
        c3 = pltpu.async_copy(w_hbm.at[pl.ds(base, epw)], w_all, ss0)

        zero = jnp.zeros((L,), jnp.float32)

        @pl.loop(0, chunk)
        def _(i):
            for j in range(d // L):
                rows0[i, pl.ds(j * L, L)] = zero

        row0 = sidx * tile_step
        for k in range(tile_win // chunk):
            pltpu.sync_copy(rows0, acc_sh.at[pl.ds(row0 + k * chunk, chunk)])
        c2.wait()
        c3.wait()
        plsc.subcore_barrier()

        def si_start(k, si_v, sem):
            pltpu.async_copy(src_hbm.at[pl.ds(base + k * chunk, chunk)], si_v, sem)

        def si_wait(k, si_v, sem):
            pltpu.make_async_copy(
                src_hbm.at[pl.ds(base + k * chunk, chunk)], si_v, sem).wait()

        def gather_start(si_v, rows_v, sem):
            pltpu.async_copy(x_hbm.at[si_v], rows_v, sem)

        def gather_wait(si_v, rows_v, sem):
            pltpu.make_async_copy(x_hbm.at[si_v], rows_v, sem).wait()

        def scat_start(k, rows_v, sem):
            pltpu.async_copy(rows_v, acc_sh.at[di_all.at[k]], sem, add=True)

        def scat_wait(k, rows_v, sem):
            pltpu.make_async_copy(rows_v, acc_sh.at[di_all.at[k]], sem).wait()

        def scale(k, rows_v):
            e0 = k * chunk

            @pl.loop(0, chunk, step=2)
            def _(e):
                for u in range(2):
                    wv = plsc.load_gather(
                        w_all, [jnp.full((L,), e0 + e + u, jnp.int32)])
                    for j in range(d // L):
                        sl = pl.ds(j * L, L)
                        rows_v[e + u, sl] = rows_v[e + u, sl] * wv

        si_start(0, si_c0, sl0)
        si_wait(0, si_c0, sl0)
        gather_start(si_c0, rows0, sg0)
        si_start(1, si_c1, sl1)
        n_pairs = (n_chunks - 1) // 2

        @pl.loop(0, n_pairs)
        def _(p):
            k0 = 2 * p
            k1 = k0 + 1

            @pl.when(p > 0)
            def _():
                scat_wait(k1 - 2, rows1, ss1)

            si_wait(k1, si_c1, sl1)
            gather_start(si_c1, rows1, sg1)
            gather_wait(si_c0, rows0, sg0)
            si_start(k0 + 2, si_c0, sl0)
            scale(k0, rows0)
            scat_start(k0, rows0, ss0)
            gather_wait(si_c1, rows1, sg1)

            @pl.when(k1 + 2 < n_chunks)
            def _():
                si_start(k1 + 2, si_c1, sl1)

            scale(k1, rows1)
            scat_start(k1, rows1, ss1)
            scat_wait(k0, rows0, ss0)
            si_wait(k0 + 2, si_c0, sl0)
            gather_start(si_c0, rows0, sg0)

        kl = n_chunks - 1
        gather_wait(si_c0, rows0, sg0)
        scale(kl, rows0)
        scat_wait(kl - 1, rows1, ss1)
        scat_start(kl, rows0, ss0)
        scat_wait(kl, rows0, ss0)

        plsc.subcore_barrier()
        pltpu.sync_copy(acc_sh.at[pl.ds(row0, tile_win)],
                        part_hbm.at[cidx, pl.ds(row0, tile_win)])

    return sc_kernel(x, src, dst3, w)


def _tc_combine_matmul_relu(partials, W):
    n_rows, d_in = partials.shape[1], partials.shape[2]
    d_out = W.shape[1]
    blk = 1000

    def body(p_ref, w_ref, o_ref):
        p = p_ref[0] + p_ref[1]
        o_ref[...] = jnp.maximum(
            jnp.dot(p, w_ref[...], preferred_element_type=jnp.float32), 0.0)

    return pl.pallas_call(
        body,
        grid=(n_rows // blk,),
        in_specs=[
            pl.BlockSpec((NC, blk, d_in), lambda i: (0, i, 0)),
            pl.BlockSpec((d_in, d_out), lambda i: (0, 0)),
        ],
        out_specs=pl.BlockSpec((blk, d_out), lambda i: (i, 0)),
        out_shape=jax.ShapeDtypeStruct((n_rows, d_out), jnp.float32),
    )(partials, W)


def kernel(x, edge_index, edge_weight, W):
    n_rows = x.shape[0]
    nw = NC * NS
    epw = edge_index.shape[1] // nw
    chunk = 80
    dst3 = edge_index[0].reshape(nw, epw // chunk, chunk)
    src = edge_index[1]
    partials = _sc_segment_sum(x, src, dst3, edge_weight, n_rows)
    return _tc_combine_matmul_relu(partials, W)


# speedup vs baseline: 10.3013x; 1.1137x over previous
import dataclasses
import functools

import jax
import jax.numpy as jnp
from jax import lax
from jax.experimental import pallas as pl
from jax.experimental.pallas import tpu as pltpu
from jax.experimental.pallas import tpu_sc as plsc

NC = 2
NS = 16
L = 16


def _sc_segment_sum(x, src, dst3, w, n_rows):
    nw = NC * NS
    e_total = src.shape[0]
    d = x.shape[1]
    epw = e_total // nw
    chunk = 80
    n_chunks = epw // chunk
    assert epw % chunk == 0 and n_chunks % 2 == 1 and dst3.shape == (nw, n_chunks, chunk)
    tile_step = (n_rows // NS) // 8 * 8
    tile_win = n_rows - tile_step * (NS - 1)
    assert tile_win % chunk == 0 and tile_win >= tile_step

    mesh = plsc.VectorSubcoreMesh(core_axis_name="c", subcore_axis_name="s")
    cp = pltpu.CompilerParams()
    if "needs_layout_passes" in pltpu.CompilerParams.__dataclass_fields__:
        cp = dataclasses.replace(cp, needs_layout_passes=False)

    @functools.partial(
        pl.kernel,
        out_type=jax.ShapeDtypeStruct((NC, n_rows, d), jnp.float32),
        mesh=mesh,
        compiler_params=cp,
        scratch_types=[
            pltpu.VMEM((n_chunks, chunk), jnp.int32),
            pltpu.VMEM((epw,), jnp.float32),
            pltpu.VMEM((chunk,), jnp.int32),
            pltpu.VMEM((chunk,), jnp.int32),
            pltpu.VMEM((chunk, d), jnp.float32),
            pltpu.VMEM((chunk, d), jnp.float32),
            pltpu.VMEM_SHARED((n_rows, d), jnp.float32),
            pltpu.SemaphoreType.DMA,
            pltpu.SemaphoreType.DMA,
            pltpu.SemaphoreType.DMA,
            pltpu.SemaphoreType.DMA,
            pltpu.SemaphoreType.DMA,
            pltpu.SemaphoreType.DMA,
        ],
    )
    def sc_kernel(x_hbm, src_hbm, dst3_hbm, w_hbm, part_hbm,
                  di_all, w_all, si_c0, si_c1, rows0, rows1, acc_sh,
                  sg0, sg1, ss0, ss1, sl0, sl1):
        cidx = lax.axis_index("c")
        sidx = lax.axis_index("s")
        wid = sidx * NC + cidx
        base = wid * epw

        c2 = pltpu.async_copy(dst3_hbm.at[wid], di_all, sg1)
        c3 = pltpu.async_copy(w_hbm.at[pl.ds(base, epw)], w_all, ss0)

        zero = jnp.zeros((L,), jnp.float32)

        @pl.loop(0, chunk)
        def _(i):
            for j in range(d // L):
                rows0[i, pl.ds(j * L, L)] = zero

        row0 = sidx * tile_step
        for k in range(tile_win // chunk):
            pltpu.sync_copy(rows0, acc_sh.at[pl.ds(row0 + k * chunk, chunk)])
        c2.wait()
        c3.wait()
        plsc.subcore_barrier()

        def si_start(k, si_v, sem):
            pltpu.async_copy(src_hbm.at[pl.ds(base + k * chunk, chunk)], si_v, sem)

        def si_wait(k, si_v, sem):
            pltpu.make_async_copy(
                src_hbm.at[pl.ds(base + k * chunk, chunk)], si_v, sem).wait()

        def gather_start(si_v, rows_v, sem):
            pltpu.async_copy(x_hbm.at[si_v], rows_v, sem)

        def gather_wait(si_v, rows_v, sem):
            pltpu.make_async_copy(x_hbm.at[si_v], rows_v, sem).wait()

        def scat_start(k, rows_v, sem):
            pltpu.async_copy(rows_v, acc_sh.at[di_all.at[k]], sem, add=True)

        def scat_wait(k, rows_v, sem):
            pltpu.make_async_copy(rows_v, acc_sh.at[di_all.at[k]], sem).wait()

        def scale(k, rows_v):
            e0 = k * chunk

            @pl.loop(0, chunk, step=L)
            def _(g):
                w16 = w_all[pl.ds(e0 + g, L)]
                for u in range(L):
                    wv = lax.gather(
                        w16, jnp.full((L, 1), u, jnp.int32),
                        lax.GatherDimensionNumbers(
                            offset_dims=(), collapsed_slice_dims=(0,),
                            start_index_map=(0,)),
                        (1,), mode=lax.GatherScatterMode.PROMISE_IN_BOUNDS)
                    for j in range(d // L):
                        sl = pl.ds(j * L, L)
                        rows_v[g + u, sl] = rows_v[g + u, sl] * wv

        si_start(0, si_c0, sl0)
        si_wait(0, si_c0, sl0)
        gather_start(si_c0, rows0, sg0)
        si_start(1, si_c1, sl1)
        n_pairs = (n_chunks - 1) // 2

        @pl.loop(0, n_pairs)
        def _(p):
            k0 = 2 * p
            k1 = k0 + 1

            @pl.when(p > 0)
            def _():
                scat_wait(k1 - 2, rows1, ss1)

            si_wait(k1, si_c1, sl1)
            gather_start(si_c1, rows1, sg1)
            gather_wait(si_c0, rows0, sg0)
            si_start(k0 + 2, si_c0, sl0)
            scale(k0, rows0)
            scat_start(k0, rows0, ss0)
            gather_wait(si_c1, rows1, sg1)

            @pl.when(k1 + 2 < n_chunks)
            def _():
                si_start(k1 + 2, si_c1, sl1)

            scale(k1, rows1)
            scat_start(k1, rows1, ss1)
            scat_wait(k0, rows0, ss0)
            si_wait(k0 + 2, si_c0, sl0)
            gather_start(si_c0, rows0, sg0)

        kl = n_chunks - 1
        gather_wait(si_c0, rows0, sg0)
        scale(kl, rows0)
        scat_wait(kl - 1, rows1, ss1)
        scat_start(kl, rows0, ss0)
        scat_wait(kl, rows0, ss0)

        plsc.subcore_barrier()
        pltpu.sync_copy(acc_sh.at[pl.ds(row0, tile_win)],
                        part_hbm.at[cidx, pl.ds(row0, tile_win)])

    return sc_kernel(x, src, dst3, w)


def _tc_combine_matmul_relu(partials, W):
    n_rows, d_in = partials.shape[1], partials.shape[2]
    d_out = W.shape[1]
    blk = 1000

    def body(p_ref, w_ref, o_ref):
        p = p_ref[0] + p_ref[1]
        o_ref[...] = jnp.maximum(
            jnp.dot(p, w_ref[...], preferred_element_type=jnp.float32), 0.0)

    return pl.pallas_call(
        body,
        grid=(n_rows // blk,),
        in_specs=[
            pl.BlockSpec((NC, blk, d_in), lambda i: (0, i, 0)),
            pl.BlockSpec((d_in, d_out), lambda i: (0, 0)),
        ],
        out_specs=pl.BlockSpec((blk, d_out), lambda i: (i, 0)),
        out_shape=jax.ShapeDtypeStruct((n_rows, d_out), jnp.float32),
    )(partials, W)


def kernel(x, edge_index, edge_weight, W):
    n_rows = x.shape[0]
    nw = NC * NS
    epw = edge_index.shape[1] // nw
    chunk = 80
    dst3 = edge_index[0].reshape(nw, epw // chunk, chunk)
    src = edge_index[1]
    partials = _sc_segment_sum(x, src, dst3, edge_weight, n_rows)
    return _tc_combine_matmul_relu(partials, W)


# speedup vs baseline: 12.5776x; 1.2210x over previous
import dataclasses
import functools

import jax
import jax.numpy as jnp
from jax import lax
from jax.experimental import pallas as pl
from jax.experimental.pallas import tpu as pltpu
from jax.experimental.pallas import tpu_sc as plsc

NC = 2
NS = 16
L = 16
NB = 3


def _sc_segment_sum(x, src, dst3, w, n_rows):
    nw = NC * NS
    e_total = src.shape[0]
    d = x.shape[1]
    epw = e_total // nw
    chunk = 80
    n_chunks = epw // chunk
    assert epw % chunk == 0 and dst3.shape == (nw, n_chunks, chunk)
    n_trips = (n_chunks - 2) // NB
    assert n_trips * NB + 2 == n_chunks
    tile_step = (n_rows // NS) // 8 * 8
    tile_win = n_rows - tile_step * (NS - 1)
    assert tile_win % chunk == 0 and tile_win >= tile_step

    mesh = plsc.VectorSubcoreMesh(core_axis_name="c", subcore_axis_name="s")
    cp = pltpu.CompilerParams()
    if "needs_layout_passes" in pltpu.CompilerParams.__dataclass_fields__:
        cp = dataclasses.replace(cp, needs_layout_passes=False)

    @functools.partial(
        pl.kernel,
        out_type=jax.ShapeDtypeStruct((NC, n_rows, d), jnp.float32),
        mesh=mesh,
        compiler_params=cp,
        scratch_types=[
            pltpu.VMEM((n_chunks, chunk), jnp.int32),
            pltpu.VMEM((NB, chunk), jnp.int32),
            pltpu.VMEM((NB, chunk), jnp.float32),
            pltpu.VMEM((chunk, d), jnp.float32),
            pltpu.VMEM((chunk, d), jnp.float32),
            pltpu.VMEM((chunk, d), jnp.float32),
            pltpu.VMEM_SHARED((n_rows, d), jnp.float32),
        ] + [pltpu.SemaphoreType.DMA] * 12,
    )
    def sc_kernel(x_hbm, src_hbm, dst3_hbm, w_hbm, part_hbm,
                  di_all, si_r, w_r, rows0, rows1, rows2, acc_sh, *sems):
        sg = sems[0:3]
        ss = sems[3:6]
        sl = sems[6:9]
        sw = sems[9:12]
        rows = (rows0, rows1, rows2)
        cidx = lax.axis_index("c")
        sidx = lax.axis_index("s")
        wid = sidx * NC + cidx
        base = wid * epw

        cdi = pltpu.async_copy(dst3_hbm.at[wid], di_all, sg[0])

        zero = jnp.zeros((L,), jnp.float32)

        @pl.loop(0, chunk)
        def _(i):
            for j in range(d // L):
                rows0[i, pl.ds(j * L, L)] = zero

        row0 = sidx * tile_step
        for k in range(tile_win // chunk):
            pltpu.sync_copy(rows0, acc_sh.at[pl.ds(row0 + k * chunk, chunk)])
        cdi.wait()
        plsc.subcore_barrier()

        def si_start(k, b):
            pltpu.async_copy(
                src_hbm.at[pl.ds(base + k * chunk, chunk)], si_r.at[b], sl[b])

        def si_wait(k, b):
            pltpu.make_async_copy(
                src_hbm.at[pl.ds(base + k * chunk, chunk)], si_r.at[b],
                sl[b]).wait()

        def w_start(k, b):
            pltpu.async_copy(
                w_hbm.at[pl.ds(base + k * chunk, chunk)], w_r.at[b], sw[b])

        def w_wait(k, b):
            pltpu.make_async_copy(
                w_hbm.at[pl.ds(base + k * chunk, chunk)], w_r.at[b],
                sw[b]).wait()

        def gather_start(b):
            pltpu.async_copy(x_hbm.at[si_r.at[b]], rows[b], sg[b])

        def gather_wait(b):
            pltpu.make_async_copy(x_hbm.at[si_r.at[b]], rows[b], sg[b]).wait()

        def scat_start(k, b):
            pltpu.async_copy(rows[b], acc_sh.at[di_all.at[k]], ss[b], add=True)

        def scat_wait(k, b):
            pltpu.make_async_copy(rows[b], acc_sh.at[di_all.at[k]], ss[b]).wait()

        def scale(k, b):
            rows_v = rows[b]

            @pl.loop(0, chunk, step=L)
            def _(g):
                w16 = w_r[b, pl.ds(g, L)]
                for u in range(L):
                    wv = lax.gather(
                        w16, jnp.full((L, 1), u, jnp.int32),
                        lax.GatherDimensionNumbers(
                            offset_dims=(), collapsed_slice_dims=(0,),
                            start_index_map=(0,)),
                        (1,), mode=lax.GatherScatterMode.PROMISE_IN_BOUNDS)
                    for j in range(d // L):
                        sl_ = pl.ds(j * L, L)
                        rows_v[g + u, sl_] = rows_v[g + u, sl_] * wv

        def body(k, b, first, issue_next):
            gather_wait(b)
            w_wait(k, b)
            scale(k, b)
            scat_start(k, b)
            if first:
                @pl.when(k > 0)
                def _():
                    scat_wait(k - 1, (b + NB - 1) % NB)
            else:
                scat_wait(k - 1, (b + NB - 1) % NB)
            if issue_next:
                b2 = (b + 2) % NB

                @pl.when(k + 2 < n_chunks)
                def _():
                    si_wait(k + 2, b2)
                    gather_start(b2)

                @pl.when(k + 3 < n_chunks)
                def _():
                    si_start(k + 3, b)
                    w_start(k + 3, b)

        si_start(0, 0)
        w_start(0, 0)
        si_start(1, 1)
        w_start(1, 1)
        si_wait(0, 0)
        gather_start(0)
        si_start(2, 2)
        w_start(2, 2)
        si_wait(1, 1)
        gather_start(1)

        @pl.loop(0, n_trips)
        def _(p):
            k0 = NB * p
            body(k0, 0, True, True)
            body(k0 + 1, 1, False, True)
            body(k0 + 2, 2, False, True)

        kl = n_chunks - 2
        body(kl, 0, False, False)
        body(kl + 1, 1, False, False)
        scat_wait(kl + 1, 1)

        plsc.subcore_barrier()
        pltpu.sync_copy(acc_sh.at[pl.ds(row0, tile_win)],
                        part_hbm.at[cidx, pl.ds(row0, tile_win)])

    return sc_kernel(x, src, dst3, w)


def _tc_combine_matmul_relu(partials, W):
    n_rows, d_in = partials.shape[1], partials.shape[2]
    d_out = W.shape[1]
    blk = 1000

    def body(p_ref, w_ref, o_ref):
        p = p_ref[0] + p_ref[1]
        o_ref[...] = jnp.maximum(
            jnp.dot(p, w_ref[...], preferred_element_type=jnp.float32), 0.0)

    return pl.pallas_call(
        body,
        grid=(n_rows // blk,),
        in_specs=[
            pl.BlockSpec((NC, blk, d_in), lambda i: (0, i, 0)),
            pl.BlockSpec((d_in, d_out), lambda i: (0, 0)),
        ],
        out_specs=pl.BlockSpec((blk, d_out), lambda i: (i, 0)),
        out_shape=jax.ShapeDtypeStruct((n_rows, d_out), jnp.float32),
    )(partials, W)


def kernel(x, edge_index, edge_weight, W):
    n_rows = x.shape[0]
    nw = NC * NS
    epw = edge_index.shape[1] // nw
    chunk = 80
    dst3 = edge_index[0].reshape(nw, epw // chunk, chunk)
    src = edge_index[1]
    partials = _sc_segment_sum(x, src, dst3, edge_weight, n_rows)
    return _tc_combine_matmul_relu(partials, W)
